# Initial kernel scaffold; baseline (speedup 1.0000x reference)
import jax
import jax.numpy as jnp
from jax.experimental import pallas as pl


def kernel(user_emb, item_emb, W_gc_0, b_gc_0, W_gc_1, b_gc_1, W_gc_2, b_gc_2, adj_val, adj_row, adj_col):
    raise NotImplementedError("write your pallas kernel here")


# speedup vs baseline: 4.3883x; 4.3883x over previous
import functools

import jax
import jax.numpy as jnp
from jax import lax
from jax.experimental import pallas as pl
from jax.experimental.pallas import tpu as pltpu
from jax.experimental.pallas import tpu_sc as plsc

N_USER = 25000
N_NODES = 50000
E = 800000
D = 64
H = 32
NS = 16
EPT = E // NS
C = 400
NCH = EPT // C
ZC = 400
RCH = N_NODES // ZC


def _spmm_body(tab_hbm, col_hbm, row_hbm, val_hbm, out_hbm,
               col_v, row_v, val_v, rows_v, acc, sem):
    c = lax.axis_index("c")
    s = lax.axis_index("s")

    def zero_row(r, carry):
        zero = jnp.zeros((16,), jnp.float32)
        for j in range(H // 16):
            rows_v[r, pl.ds(j * 16, 16)] = zero
        return carry

    lax.fori_loop(0, ZC, zero_row, 0)

    for z in range((RCH + NS - 1) // NS):
        k = s + NS * z

        @pl.when(k < RCH)
        def _():
            pltpu.sync_copy(rows_v, acc.at[pl.ds(k * ZC, ZC)])

    plsc.subcore_barrier()

    base = s * EPT
    coff = c * N_NODES

    def chunk(g, carry):
        off = base + g * C
        pltpu.sync_copy(col_hbm.at[pl.ds(off, C)], col_v)
        pltpu.sync_copy(row_hbm.at[pl.ds(off, C)], row_v)
        pltpu.sync_copy(val_hbm.at[pl.ds(off, C)], val_v)

        def shift(b, carry2):
            col_v[pl.ds(b * 16, 16)] = col_v[pl.ds(b * 16, 16)] + coff
            return carry2

        lax.fori_loop(0, C // 16, shift, 0)

        pltpu.async_copy(tab_hbm.at[col_v], rows_v, sem).wait()

        def scale(e, carry2):
            sp = jnp.full((16,), e, jnp.int32)
            v16 = plsc.load_gather(val_v, [sp])
            for j in range(H // 16):
                x = rows_v[e, pl.ds(j * 16, 16)]
                rows_v[e, pl.ds(j * 16, 16)] = x * v16
            return carry2

        lax.fori_loop(0, C, scale, 0)

        pltpu.sync_copy(rows_v, acc.at[row_v], add=True)
        return carry

    lax.fori_loop(0, NCH, chunk, 0)

    plsc.subcore_barrier()

    for z in range((RCH + NS - 1) // NS):
        k = s + NS * z

        @pl.when(k < RCH)
        def _():
            pltpu.sync_copy(acc.at[pl.ds(k * ZC, ZC)],
                            out_hbm.at[c, pl.ds(k * ZC, ZC)])


def _spmm(tab, col, row, val):
    mesh = plsc.VectorSubcoreMesh(core_axis_name="c", subcore_axis_name="s")
    f = pl.kernel(
        _spmm_body,
        out_type=jax.ShapeDtypeStruct((2, N_NODES, H), jnp.float32),
        mesh=mesh,
        compiler_params=pltpu.CompilerParams(needs_layout_passes=False,
                                             use_tc_tiling_on_sc=False),
        scratch_types=[
            pltpu.VMEM((C,), jnp.int32),
            pltpu.VMEM((C,), jnp.int32),
            pltpu.VMEM((C,), jnp.float32),
            pltpu.VMEM((C, H), jnp.float32),
            pltpu.VMEM_SHARED((N_NODES, H), jnp.float32),
            pltpu.SemaphoreType.DMA,
        ],
    )
    return f(tab, col, row, val)


def _mm_body(last, a_ref, w_ref, b_ref, o_ref):
    w = w_ref[...]
    y = (jnp.dot(a_ref[0], w[:H, :], preferred_element_type=jnp.float32)
         + jnp.dot(a_ref[1], w[H:, :], preferred_element_type=jnp.float32)
         + b_ref[...])
    y = jnp.maximum(y, 0.2 * y)
    if last:
        o_ref[...] = y
    else:
        o_ref[0] = y[:, :H]
        o_ref[1] = y[:, H:]


def _mm(a, w, b, last):
    R = 10000
    if last:
        out_specs = pl.BlockSpec((R, D), lambda i: (i, 0))
        out_shape = jax.ShapeDtypeStruct((N_NODES, D), jnp.float32)
    else:
        out_specs = pl.BlockSpec((2, R, H), lambda i: (0, i, 0))
        out_shape = jax.ShapeDtypeStruct((2, N_NODES, H), jnp.float32)
    return pl.pallas_call(
        functools.partial(_mm_body, last),
        grid=(N_NODES // R,),
        in_specs=[
            pl.BlockSpec((2, R, H), lambda i: (0, i, 0)),
            pl.BlockSpec((D, D), lambda i: (0, 0)),
            pl.BlockSpec((1, D), lambda i: (0, 0)),
        ],
        out_specs=out_specs,
        out_shape=out_shape,
    )(a, w, b)


def kernel(user_emb, item_emb, W_gc_0, b_gc_0, W_gc_1, b_gc_1, W_gc_2, b_gc_2,
           adj_val, adj_row, adj_col):
    emb = jnp.concatenate([user_emb, item_emb], axis=0)
    t = jnp.stack([emb[:, :H], emb[:, H:]], axis=0)
    col = adj_col.astype(jnp.int32)
    row = adj_row.astype(jnp.int32)
    val = adj_val
    Ws = [W_gc_0, W_gc_1, W_gc_2]
    bs = [b_gc_0, b_gc_1, b_gc_2]
    for k in range(2):
        a = _spmm(t.reshape(2 * N_NODES, H), col, row, val)
        t = _mm(a, Ws[k], bs[k], last=False)
    a = _spmm(t.reshape(2 * N_NODES, H), col, row, val)
    y = _mm(a, Ws[2], bs[2], last=True)
    return y[:N_USER], y[N_USER:]
